# baseline (speedup 1.0000x reference)
import functools

import jax
import jax.numpy as jnp
from jax import lax
from jax.experimental import pallas as pl
from jax.experimental.pallas import tpu as pltpu
from jax.experimental.pallas import tpu_sc as plsc


def kernel(x, edge_index, expander_edges, virt_table):
    num_node = x.shape[0]
    num_virt = virt_table.shape[0]
    n_edges = edge_index.shape[1]
    edtype = edge_index.dtype

    ve_cols = 2 * num_virt * num_node
    lane_t = 128

    n_chunk = 16
    cw = -(-n_edges // n_chunk // lane_t) * lane_t
    cw_last = n_edges - (n_chunk - 1) * cw

    vw_a = 39 * lane_t
    vw_b = 40 * lane_t
    n_a = 30

    mesh = plsc.VectorSubcoreMesh(core_axis_name="c", subcore_axis_name="s")

    @functools.partial(
        pl.kernel,
        out_type=(
            jax.ShapeDtypeStruct((2, 2 * n_edges), edtype),
            jax.ShapeDtypeStruct(virt_table.shape, virt_table.dtype),
            jax.ShapeDtypeStruct((2, ve_cols), edtype),
        ),
        mesh=mesh,
        scratch_types=[
            pltpu.VMEM((2, vw_b), jnp.int32),
            pltpu.SemaphoreType.DMA,
            pltpu.SemaphoreType.DMA,
        ],
    )
    def sc_kernel(edge_hbm, exp_hbm, virt_hbm, out_e, out_v, out_ve,
                  buf, csem, vsem):
        wid = lax.axis_index("s") * 2 + lax.axis_index("c")

        h = wid // n_chunk
        c = wid % n_chunk

        def concat_cases(action):
            for hv, src_ref in ((0, edge_hbm), (1, exp_hbm)):
                for last in (False, True):
                    w = cw_last if last else cw
                    cond = (h == hv) & ((c == n_chunk - 1) if last
                                        else (c < n_chunk - 1))

                    @pl.when(cond)
                    def _(src_ref=src_ref, hv=hv, w=w):
                        col = pl.multiple_of(c * cw, lane_t)
                        dcol = pl.multiple_of(hv * n_edges + c * cw, lane_t)
                        action(src_ref.at[:, pl.ds(col, w)],
                               out_e.at[:, pl.ds(dcol, w)])

        concat_cases(lambda s, d: pltpu.make_async_copy(s, d, csem).start())

        @pl.when(wid == 0)
        def _():
            pltpu.make_async_copy(virt_hbm, out_v, vsem).start()

        iota = lax.iota(jnp.int32, 16)
        off = jnp.where(wid < n_a, wid * vw_a,
                        n_a * vw_a + (wid - n_a) * vw_b)

        def gen(w):
            def body(k, carry):
                b = off + k * 16
                seg = b // num_node
                within = b - seg * num_node
                ar = within + iota
                cv = jnp.full((16,), num_node, jnp.int32) + (seg >> 1)
                even = (seg & 1) == 0
                buf[0, pl.ds(k * 16, 16)] = jnp.where(even, ar, cv)
                buf[1, pl.ds(k * 16, 16)] = jnp.where(even, cv, ar)
                return carry

            lax.fori_loop(0, w // 16, body, 0, unroll=4)
            dst = out_ve.at[:, pl.ds(pl.multiple_of(off, lane_t), w)]
            pltpu.sync_copy(buf.at[:, pl.ds(0, w)], dst)

        @pl.when(wid < n_a)
        def _():
            gen(vw_a)

        @pl.when(wid >= n_a)
        def _():
            gen(vw_b)

        concat_cases(lambda s, d: pltpu.make_async_copy(s, d, csem).wait())

        @pl.when(wid == 0)
        def _():
            pltpu.make_async_copy(virt_hbm, out_v, vsem).wait()

    return sc_kernel(edge_index, expander_edges, virt_table)


# speedup vs baseline: 1.0020x; 1.0020x over previous
import functools

import jax
import jax.numpy as jnp
from jax import lax
from jax.experimental import pallas as pl
from jax.experimental.pallas import tpu as pltpu
from jax.experimental.pallas import tpu_sc as plsc


def kernel(x, edge_index, expander_edges, virt_table):
    num_node = x.shape[0]
    num_virt = virt_table.shape[0]
    n_edges = edge_index.shape[1]
    edtype = edge_index.dtype

    ve_cols = 2 * num_virt * num_node
    lane_t = 128

    n_chunk = 16
    cw = -(-n_edges // n_chunk // lane_t) * lane_t
    cw_last = n_edges - (n_chunk - 1) * cw

    vw_a = 39 * lane_t
    vw_b = 40 * lane_t
    n_a = 30

    mesh = plsc.VectorSubcoreMesh(core_axis_name="c", subcore_axis_name="s")

    @functools.partial(
        pl.kernel,
        out_type=(
            jax.ShapeDtypeStruct((2, 2 * n_edges), edtype),
            jax.ShapeDtypeStruct(virt_table.shape, virt_table.dtype),
            jax.ShapeDtypeStruct((2, ve_cols), edtype),
        ),
        mesh=mesh,
        scratch_types=[
            pltpu.VMEM((2, vw_b), jnp.int32),
            pltpu.SemaphoreType.DMA,
            pltpu.SemaphoreType.DMA,
        ],
    )
    def sc_kernel(edge_hbm, exp_hbm, virt_hbm, out_e, out_v, out_ve,
                  buf, csem, vsem):
        wid = lax.axis_index("s") * 2 + lax.axis_index("c")

        h = wid // n_chunk
        c = wid % n_chunk

        def concat_cases(action):
            for hv, src_ref in ((0, edge_hbm), (1, exp_hbm)):
                for last in (False, True):
                    w = cw_last if last else cw
                    cond = (h == hv) & ((c == n_chunk - 1) if last
                                        else (c < n_chunk - 1))

                    @pl.when(cond)
                    def _(src_ref=src_ref, hv=hv, w=w):
                        col = pl.multiple_of(c * cw, lane_t)
                        dcol = pl.multiple_of(hv * n_edges + c * cw, lane_t)
                        action(src_ref.at[:, pl.ds(col, w)],
                               out_e.at[:, pl.ds(dcol, w)])

        concat_cases(lambda s, d: pltpu.make_async_copy(s, d, csem).start())

        @pl.when(wid == 0)
        def _():
            pltpu.make_async_copy(virt_hbm, out_v, vsem).start()

        iota = lax.iota(jnp.int32, 16)
        off = jnp.where(wid < n_a, wid * vw_a,
                        n_a * vw_a + (wid - n_a) * vw_b)
        seg0 = off // num_node
        within0 = off - seg0 * num_node

        def gen(w):
            def body(k, carry):
                seg, within = carry
                ar = within + iota
                cv = jnp.full((16,), num_node, jnp.int32) + (seg >> 1)
                even = (seg & 1) == 0
                buf[0, pl.ds(k * 16, 16)] = jnp.where(even, ar, cv)
                buf[1, pl.ds(k * 16, 16)] = jnp.where(even, cv, ar)
                within = within + 16
                wrap = within >= num_node
                seg = jnp.where(wrap, seg + 1, seg)
                within = jnp.where(wrap, 0, within)
                return (seg, within)

            lax.fori_loop(0, w // 16, body, (seg0, within0), unroll=4)
            dst = out_ve.at[:, pl.ds(pl.multiple_of(off, lane_t), w)]
            pltpu.sync_copy(buf.at[:, pl.ds(0, w)], dst)

        @pl.when(wid < n_a)
        def _():
            gen(vw_a)

        @pl.when(wid >= n_a)
        def _():
            gen(vw_b)

        concat_cases(lambda s, d: pltpu.make_async_copy(s, d, csem).wait())

        @pl.when(wid == 0)
        def _():
            pltpu.make_async_copy(virt_hbm, out_v, vsem).wait()

    return sc_kernel(edge_index, expander_edges, virt_table)


# speedup vs baseline: 7.0668x; 7.0525x over previous
import functools

import jax
import jax.numpy as jnp
from jax import lax
from jax.experimental import pallas as pl
from jax.experimental.pallas import tpu as pltpu
from jax.experimental.pallas import tpu_sc as plsc


def kernel(x, edge_index, expander_edges, virt_table):
    num_node = x.shape[0]
    num_virt = virt_table.shape[0]
    n_edges = edge_index.shape[1]
    edtype = edge_index.dtype

    ve_cols = 2 * num_virt * num_node
    lane_t = 128

    n_chunk = 16
    cw = -(-n_edges // n_chunk // lane_t) * lane_t
    cw_last = n_edges - (n_chunk - 1) * cw

    vw_a = 39 * lane_t
    vw_b = 40 * lane_t
    n_a = 30

    mesh = plsc.VectorSubcoreMesh(core_axis_name="c", subcore_axis_name="s")

    @functools.partial(
        pl.kernel,
        out_type=(
            jax.ShapeDtypeStruct((2, 2 * n_edges), edtype),
            jax.ShapeDtypeStruct(virt_table.shape, virt_table.dtype),
            jax.ShapeDtypeStruct((2, ve_cols), edtype),
        ),
        mesh=mesh,
        scratch_types=[
            pltpu.VMEM((2, vw_b), jnp.int32),
            pltpu.VMEM((2, cw), jnp.int32),
            pltpu.SemaphoreType.DMA,
            pltpu.SemaphoreType.DMA,
            pltpu.SemaphoreType.DMA,
        ],
    )
    def sc_kernel(edge_hbm, exp_hbm, virt_hbm, out_e, out_v, out_ve,
                  buf, ebuf, csem, wsem, vsem):
        wid = lax.axis_index("s") * 2 + lax.axis_index("c")

        h = wid // n_chunk
        c = wid % n_chunk

        def concat_cases(action):
            for hv, src_ref in ((0, edge_hbm), (1, exp_hbm)):
                for last in (False, True):
                    w = cw_last if last else cw
                    cond = (h == hv) & ((c == n_chunk - 1) if last
                                        else (c < n_chunk - 1))

                    @pl.when(cond)
                    def _(src_ref=src_ref, hv=hv, w=w):
                        col = pl.multiple_of(c * cw, lane_t)
                        dcol = pl.multiple_of(hv * n_edges + c * cw, lane_t)
                        action(src_ref.at[:, pl.ds(col, w)],
                               ebuf.at[:, pl.ds(0, w)],
                               out_e.at[:, pl.ds(dcol, w)])

        concat_cases(lambda s, b, d: pltpu.make_async_copy(s, b, csem).start())

        @pl.when(wid == 0)
        def _():
            pltpu.make_async_copy(virt_hbm, out_v, vsem).start()

        iota = lax.iota(jnp.int32, 16)
        off = jnp.where(wid < n_a, wid * vw_a,
                        n_a * vw_a + (wid - n_a) * vw_b)
        seg0 = off // num_node
        within0 = off - seg0 * num_node

        def gen(w):
            def body(k, carry):
                seg, within = carry
                ar = within + iota
                cv = jnp.full((16,), num_node, jnp.int32) + (seg >> 1)
                even = (seg & 1) == 0
                buf[0, pl.ds(k * 16, 16)] = jnp.where(even, ar, cv)
                buf[1, pl.ds(k * 16, 16)] = jnp.where(even, cv, ar)
                within = within + 16
                wrap = within >= num_node
                seg = jnp.where(wrap, seg + 1, seg)
                within = jnp.where(wrap, 0, within)
                return (seg, within)

            lax.fori_loop(0, w // 16, body, (seg0, within0), unroll=4)
            dst = out_ve.at[:, pl.ds(pl.multiple_of(off, lane_t), w)]
            pltpu.sync_copy(buf.at[:, pl.ds(0, w)], dst)

        @pl.when(wid < n_a)
        def _():
            gen(vw_a)

        @pl.when(wid >= n_a)
        def _():
            gen(vw_b)

        concat_cases(lambda s, b, d: pltpu.make_async_copy(s, b, csem).wait())
        concat_cases(lambda s, b, d: pltpu.make_async_copy(b, d, wsem).start())
        concat_cases(lambda s, b, d: pltpu.make_async_copy(b, d, wsem).wait())

        @pl.when(wid == 0)
        def _():
            pltpu.make_async_copy(virt_hbm, out_v, vsem).wait()

    return sc_kernel(edge_index, expander_edges, virt_table)


# speedup vs baseline: 7.2093x; 1.0202x over previous
import functools

import jax
import jax.numpy as jnp
from jax import lax
from jax.experimental import pallas as pl
from jax.experimental.pallas import tpu as pltpu
from jax.experimental.pallas import tpu_sc as plsc

_NW = 32
_LT = 128


def kernel(x, edge_index, expander_edges, virt_table):
    num_node = x.shape[0]
    num_virt = virt_table.shape[0]
    n_edges = edge_index.shape[1]
    edtype = edge_index.dtype

    ve_cols = 2 * num_virt * num_node

    cw = -(-n_edges // _NW // _LT) * _LT
    c_excess = _NW * cw - n_edges

    vw = -(-ve_cols // _NW // _LT) * _LT
    v_excess = _NW * vw - ve_cols

    mesh = plsc.VectorSubcoreMesh(core_axis_name="c", subcore_axis_name="s")

    @functools.partial(
        pl.kernel,
        out_type=(
            jax.ShapeDtypeStruct((2, 2 * n_edges), edtype),
            jax.ShapeDtypeStruct(virt_table.shape, virt_table.dtype),
            jax.ShapeDtypeStruct((2, ve_cols), edtype),
        ),
        mesh=mesh,
        scratch_types=[
            pltpu.VMEM((2, vw), jnp.int32),
            pltpu.VMEM((2, cw), jnp.int32),
            pltpu.VMEM((2, cw), jnp.int32),
            pltpu.SemaphoreType.DMA,
            pltpu.SemaphoreType.DMA,
            pltpu.SemaphoreType.DMA,
            pltpu.SemaphoreType.DMA,
        ],
    )
    def sc_kernel(edge_hbm, exp_hbm, virt_hbm, out_e, out_v, out_ve,
                  buf, ebuf, xbuf, csem, wsem, vesem, vsem):
        wid = lax.axis_index("s") * 2 + lax.axis_index("c")

        col = pl.multiple_of(wid * cw - (wid // (_NW - 1)) * c_excess, _LT)
        e_src = edge_hbm.at[:, pl.ds(col, cw)]
        x_src = exp_hbm.at[:, pl.ds(col, cw)]
        pltpu.make_async_copy(e_src, ebuf, csem).start()
        pltpu.make_async_copy(x_src, xbuf, csem).start()

        @pl.when(wid == 0)
        def _():
            pltpu.make_async_copy(virt_hbm, out_v, vsem).start()

        iota = lax.iota(jnp.int32, 16)
        off = pl.multiple_of(wid * vw - (wid // (_NW - 1)) * v_excess, _LT)
        seg0 = off // num_node
        within0 = off - seg0 * num_node

        def body(k, carry):
            seg, within = carry
            ar = within + iota
            cv = jnp.full((16,), num_node, jnp.int32) + (seg >> 1)
            even = (seg & 1) == 0
            buf[0, pl.ds(k * 16, 16)] = jnp.where(even, ar, cv)
            buf[1, pl.ds(k * 16, 16)] = jnp.where(even, cv, ar)
            within = within + 16
            wrap = within >= num_node
            seg = jnp.where(wrap, seg + 1, seg)
            within = jnp.where(wrap, 0, within)
            return (seg, within)

        lax.fori_loop(0, vw // 16, body, (seg0, within0), unroll=4)
        ve_dst = out_ve.at[:, pl.ds(off, vw)]
        pltpu.make_async_copy(buf, ve_dst, vesem).start()

        e_dst = out_e.at[:, pl.ds(col, cw)]
        x_dst = out_e.at[:, pl.ds(pl.multiple_of(n_edges + col, _LT), cw)]
        pltpu.make_async_copy(e_src, ebuf, csem).wait()
        pltpu.make_async_copy(x_src, xbuf, csem).wait()
        pltpu.make_async_copy(ebuf, e_dst, wsem).start()
        pltpu.make_async_copy(xbuf, x_dst, wsem).start()

        pltpu.make_async_copy(buf, ve_dst, vesem).wait()
        pltpu.make_async_copy(ebuf, e_dst, wsem).wait()
        pltpu.make_async_copy(xbuf, x_dst, wsem).wait()

        @pl.when(wid == 0)
        def _():
            pltpu.make_async_copy(virt_hbm, out_v, vsem).wait()

    return sc_kernel(edge_index, expander_edges, virt_table)


# speedup vs baseline: 7.2194x; 1.0014x over previous
import functools

import jax
import jax.numpy as jnp
from jax import lax
from jax.experimental import pallas as pl
from jax.experimental.pallas import tpu as pltpu
from jax.experimental.pallas import tpu_sc as plsc

_NW = 32
_LT = 128


def kernel(x, edge_index, expander_edges, virt_table):
    num_node = x.shape[0]
    num_virt = virt_table.shape[0]
    n_edges = edge_index.shape[1]
    edtype = edge_index.dtype

    ve_cols = 2 * num_virt * num_node

    cw = -(-n_edges // _NW // _LT) * _LT
    c_excess = _NW * cw - n_edges

    vw = -(-ve_cols // _NW // _LT) * _LT
    v_excess = _NW * vw - ve_cols

    mesh = plsc.VectorSubcoreMesh(core_axis_name="c", subcore_axis_name="s")

    @functools.partial(
        pl.kernel,
        out_type=(
            jax.ShapeDtypeStruct((2, 2 * n_edges), edtype),
            jax.ShapeDtypeStruct(virt_table.shape, virt_table.dtype),
            jax.ShapeDtypeStruct((2, ve_cols), edtype),
        ),
        mesh=mesh,
        scratch_types=[
            pltpu.VMEM((2, vw), jnp.int32),
            pltpu.VMEM((2, cw), jnp.int32),
            pltpu.VMEM((2, cw), jnp.int32),
            pltpu.SemaphoreType.DMA,
            pltpu.SemaphoreType.DMA,
            pltpu.SemaphoreType.DMA,
            pltpu.SemaphoreType.DMA,
        ],
    )
    def sc_kernel(edge_hbm, exp_hbm, virt_hbm, out_e, out_v, out_ve,
                  buf, ebuf, xbuf, csem, wsem, vesem, vsem):
        wid = lax.axis_index("s") * 2 + lax.axis_index("c")

        col = pl.multiple_of(wid * cw - (wid // (_NW - 1)) * c_excess, _LT)
        e_src = edge_hbm.at[:, pl.ds(col, cw)]
        x_src = exp_hbm.at[:, pl.ds(col, cw)]
        pltpu.make_async_copy(e_src, ebuf, csem).start()
        pltpu.make_async_copy(x_src, xbuf, csem).start()

        @pl.when(wid == 0)
        def _():
            pltpu.make_async_copy(virt_hbm, out_v, vsem).start()

        iota = lax.iota(jnp.int32, 16)
        off = pl.multiple_of(wid * vw - (wid // (_NW - 1)) * v_excess, _LT)
        seg0 = off // num_node
        within0 = off - seg0 * num_node

        def body(k, carry):
            seg, within = carry
            ar = within + iota
            cv = jnp.full((16,), num_node, jnp.int32) + (seg >> 1)
            even = (seg & 1) == 0
            buf[0, pl.ds(k * 16, 16)] = jnp.where(even, ar, cv)
            buf[1, pl.ds(k * 16, 16)] = jnp.where(even, cv, ar)
            within = within + 16
            wrap = within >= num_node
            seg = jnp.where(wrap, seg + 1, seg)
            within = jnp.where(wrap, 0, within)
            return (seg, within)

        e_dst = out_e.at[:, pl.ds(col, cw)]
        x_dst = out_e.at[:, pl.ds(pl.multiple_of(n_edges + col, _LT), cw)]
        pltpu.make_async_copy(e_src, ebuf, csem).wait()
        pltpu.make_async_copy(x_src, xbuf, csem).wait()
        pltpu.make_async_copy(ebuf, e_dst, wsem).start()
        pltpu.make_async_copy(xbuf, x_dst, wsem).start()

        lax.fori_loop(0, vw // 16, body, (seg0, within0), unroll=8)
        ve_dst = out_ve.at[:, pl.ds(off, vw)]
        pltpu.make_async_copy(buf, ve_dst, vesem).start()

        pltpu.make_async_copy(buf, ve_dst, vesem).wait()
        pltpu.make_async_copy(ebuf, e_dst, wsem).wait()
        pltpu.make_async_copy(xbuf, x_dst, wsem).wait()

        @pl.when(wid == 0)
        def _():
            pltpu.make_async_copy(virt_hbm, out_v, vsem).wait()

    return sc_kernel(edge_index, expander_edges, virt_table)
